# Initial kernel scaffold; baseline (speedup 1.0000x reference)
import jax
import jax.numpy as jnp
from jax.experimental import pallas as pl


def kernel(documents, ent_desc, doc_lens, ent_lens, adj_lists, feature_lists, sentPerDoc, word_emb, topic_emb, W_text, b_text, W_topic, W_gat, a_src, a_dst, W_cls, b_cls):
    raise NotImplementedError("write your pallas kernel here")


# speedup vs baseline: 5.1313x; 5.1313x over previous
import functools

import jax
import jax.numpy as jnp
from jax import lax
from jax.experimental import pallas as pl
from jax.experimental.pallas import tpu as pltpu, tpu_sc as plsc

N_SENT = 10000
N_TOPIC = 100
N_NODES = N_SENT + N_TOPIC
N_EDGES = 320000
EMB = 128
NDOCS = 500
L_TOK = 50

NC = 2
NS = 16
NW = NC * NS

HEXT = 144
TOK_PAD = 512000
CHUNKS1 = TOK_PAD // 128
CH1_PER_W = CHUNKS1 // NW
SENT_PER_W = 320
ACC1_ROWS = 10112
ACC3_ROWS = 10112
CHUNKS3 = N_EDGES // 128
CH3_PER_W = 79

@functools.cache
def _mesh():
    return plsc.VectorSubcoreMesh(core_axis_name="c", subcore_axis_name="s",
                                  num_cores=NC, num_subcores=NS)


def _s1_body(docs_ref, lens_ref, emb_ref, z_ref, out_ref,
             acc, idxb, dstb, rowsb, lensb, sem):
    c = lax.axis_index("c")
    s = lax.axis_index("s")
    wid = s * NC + c
    trash = N_SENT + s

    pltpu.sync_copy(z_ref, acc.at[pl.ds(s * 632, 632)])
    pltpu.sync_copy(lens_ref.at[pl.ds(wid * SENT_PER_W, SENT_PER_W)], lensb)
    plsc.subcore_barrier()

    base_sent = wid * SENT_PER_W

    @pl.loop(0, CH1_PER_W)
    def _chunk(i):
        off = (wid * CH1_PER_W + i) * 128
        pltpu.sync_copy(docs_ref.at[pl.ds(off, 128)], idxb)
        pltpu.async_copy(emb_ref.at[idxb], rowsb, sem).wait()
        for g in range(8):
            pos = off + g * 16 + lax.iota(jnp.int32, 16)
            sent = pos // L_TOK
            j = pos - sent * L_TOK
            ln = plsc.load_gather(lensb, [sent - base_sent])
            ln = jnp.maximum(ln, 1)
            valid = (j < ln) & (sent < N_SENT)
            dstb[pl.ds(g * 16, 16)] = jnp.where(valid, sent, trash)
        pltpu.sync_copy(rowsb, acc.at[dstb], add=True)

    plsc.subcore_barrier()
    pltpu.sync_copy(acc.at[pl.ds(s * 632, 632)],
                    out_ref.at[c, pl.ds(s * 632, 632)])


@functools.cache
def _stage1():
    return pl.kernel(
        _s1_body,
        out_type=jax.ShapeDtypeStruct((NC, ACC1_ROWS, EMB), jnp.float32),
        mesh=_mesh(),
        compiler_params=pltpu.CompilerParams(
            use_tc_tiling_on_sc=False, needs_layout_passes=False),
        scratch_types=[
            pltpu.VMEM_SHARED((ACC1_ROWS, EMB), jnp.float32),
            pltpu.VMEM((128,), jnp.int32),
            pltpu.VMEM((128,), jnp.int32),
            pltpu.VMEM((128, EMB), jnp.float32),
            pltpu.VMEM((SENT_PER_W,), jnp.int32),
            pltpu.SemaphoreType.DMA,
        ],
    )


def _s2_body(sums_ref, lens_ref, Wt_ref, bt_ref, fl_ref, te_ref, Wp_ref,
             Wg_ref, asrc_ref, adst_ref, X_ref, hext_ref, dv_ref):
    f32 = jnp.float32
    sums = sums_ref[0] + sums_ref[1]
    lens = jnp.maximum(lens_ref[...].astype(f32), 1.0)
    pooled = sums[:N_SENT] / lens
    d = jnp.maximum(
        jnp.dot(pooled, Wt_ref[...], preferred_element_type=f32)
        + bt_ref[...], 0.0)
    P = (fl_ref[...] ==
         lax.broadcasted_iota(jnp.int32, (N_TOPIC, N_TOPIC), 1)).astype(f32)
    t = jnp.dot(jnp.dot(P, te_ref[...], preferred_element_type=f32),
                Wp_ref[...], preferred_element_type=f32)
    X = jnp.concatenate([d, t], axis=0)
    h = jnp.dot(X, Wg_ref[...], preferred_element_type=f32)
    sc = jnp.dot(h, asrc_ref[...], preferred_element_type=f32)
    dv = jnp.dot(h, adst_ref[...], preferred_element_type=f32)
    X_ref[...] = X
    dv_ref[...] = dv
    ones = jnp.ones((N_NODES, 1), f32)
    zpad = jnp.zeros((N_NODES, HEXT - EMB - 3), f32)
    hext_ref[...] = jnp.concatenate([h, ones, sc, dv, zpad], axis=1)


_stage2 = pl.pallas_call(
    _s2_body,
    out_shape=[
        jax.ShapeDtypeStruct((N_NODES, EMB), jnp.float32),
        jax.ShapeDtypeStruct((N_NODES, HEXT), jnp.float32),
        jax.ShapeDtypeStruct((N_NODES, 1), jnp.float32),
    ],
)


def _s3_body(src_ref, dst_ref, hext_ref, dv_ref, z_ref, out_ref,
             acc, sidxb, didxb, rowsb, dvb, exb, sem):
    c = lax.axis_index("c")
    s = lax.axis_index("s")
    wid = s * NC + c

    pltpu.sync_copy(z_ref, acc.at[pl.ds(s * 632, 632)])
    plsc.subcore_barrier()

    @pl.loop(0, CH3_PER_W)
    def _chunk(i):
        cidx = wid + NW * i

        @pl.when(cidx < CHUNKS3)
        def _():
            off = cidx * 128
            pltpu.sync_copy(src_ref.at[pl.ds(off, 128)], sidxb)
            pltpu.sync_copy(dst_ref.at[pl.ds(off, 128)], didxb)
            pltpu.async_copy(hext_ref.at[sidxb], rowsb, sem).wait()
            pltpu.async_copy(dv_ref.at[didxb], dvb, sem).wait()
            for g in range(8):
                ii = lax.iota(jnp.int32, 16) + g * 16
                ssrc = plsc.load_gather(rowsb, [ii, jnp.full((16,), EMB + 1,
                                                             jnp.int32)])
                e = ssrc + dvb[pl.ds(g * 16, 16)]
                e = jnp.maximum(e, 0.2 * e)
                exb[pl.ds(g * 16, 16)] = jnp.exp(e)

            @pl.loop(0, 128)
            def _edge(ei):
                sp = plsc.load_gather(exb, [jnp.full((16,), ei, jnp.int32)])
                for cc in range(HEXT // 16):
                    rowsb[ei, pl.ds(cc * 16, 16)] = (
                        rowsb[ei, pl.ds(cc * 16, 16)] * sp)

            pltpu.sync_copy(rowsb, acc.at[didxb], add=True)

    plsc.subcore_barrier()
    pltpu.sync_copy(acc.at[pl.ds(s * 632, 632)],
                    out_ref.at[c, pl.ds(s * 632, 632)])


@functools.cache
def _stage3():
    return pl.kernel(
        _s3_body,
        out_type=jax.ShapeDtypeStruct((NC, ACC3_ROWS, HEXT), jnp.float32),
        mesh=_mesh(),
        compiler_params=pltpu.CompilerParams(
            use_tc_tiling_on_sc=False, needs_layout_passes=False),
        scratch_types=[
            pltpu.VMEM_SHARED((ACC3_ROWS, HEXT), jnp.float32),
            pltpu.VMEM((128,), jnp.int32),
            pltpu.VMEM((128,), jnp.int32),
            pltpu.VMEM((128, HEXT), jnp.float32),
            pltpu.VMEM((128,), jnp.float32),
            pltpu.VMEM((128,), jnp.float32),
            pltpu.SemaphoreType.DMA,
        ],
    )


def _s4_body(msg_ref, X_ref, spd_ref, Wc_ref, bc_ref, out_ref):
    f32 = jnp.float32
    m = msg_ref[0] + msg_ref[1]
    msg = m[:N_NODES, :EMB]
    den = m[:N_NODES, EMB:EMB + 1]
    o = msg / jnp.maximum(den, 1e-9)
    o = jnp.where(o > 0, o, jnp.exp(o) - 1.0) + X_ref[...]
    Xs = o[:N_SENT]
    oh = (spd_ref[...] ==
          lax.broadcasted_iota(jnp.int32, (N_SENT, NDOCS), 1)).astype(f32)
    sums = lax.dot_general(oh, Xs, (((0,), (0,)), ((), ())),
                           preferred_element_type=f32)
    cnts = jnp.sum(oh, axis=0)[:, None]
    Xd = sums / jnp.maximum(cnts, 1.0)
    logits = jnp.dot(Xd, Wc_ref[...], preferred_element_type=f32) + bc_ref[...]
    out_ref[...] = jax.nn.softmax(logits, axis=1)


_stage4 = pl.pallas_call(
    _s4_body,
    out_shape=jax.ShapeDtypeStruct((NDOCS, 5), jnp.float32),
)


@jax.jit
def kernel(documents, ent_desc, doc_lens, ent_lens, adj_lists, feature_lists,
           sentPerDoc, word_emb, topic_emb, W_text, b_text, W_topic, W_gat,
           a_src, a_dst, W_cls, b_cls):
    i32 = jnp.int32
    f32 = jnp.float32
    docs_flat = jnp.pad(documents.astype(i32).reshape(-1),
                        (0, TOK_PAD - N_SENT * L_TOK))
    lens_pad = jnp.pad(doc_lens.astype(i32), (0, NW * SENT_PER_W - N_SENT))
    z1 = jnp.zeros((632, EMB), f32)
    sums = _stage1()(docs_flat, lens_pad, word_emb.astype(f32), z1)

    X, hext, dv = _stage2(
        sums, doc_lens.astype(i32).reshape(-1, 1), W_text.astype(f32),
        b_text.astype(f32).reshape(1, EMB), feature_lists.astype(i32).reshape(-1, 1),
        topic_emb.astype(f32), W_topic.astype(f32), W_gat.astype(f32),
        a_src.astype(f32).reshape(EMB, 1), a_dst.astype(f32).reshape(EMB, 1))

    src = adj_lists[0].astype(i32)
    dst = adj_lists[1].astype(i32)
    z3 = jnp.zeros((632, HEXT), f32)
    msg = _stage3()(src, dst, hext, dv.reshape(-1), z3)

    return _stage4(msg, X, sentPerDoc.astype(i32).reshape(-1, 1),
                   W_cls.astype(f32), b_cls.astype(f32).reshape(1, 5))


# speedup vs baseline: 5.5905x; 1.0895x over previous
import functools

import jax
import jax.numpy as jnp
from jax import lax
from jax.experimental import pallas as pl
from jax.experimental.pallas import tpu as pltpu, tpu_sc as plsc

N_SENT = 10000
N_TOPIC = 100
N_NODES = N_SENT + N_TOPIC
N_EDGES = 320000
EMB = 128
NDOCS = 500
L_TOK = 50

NC = 2
NS = 16
NW = NC * NS

SENT_PER_W = 320
SENT_PAD = NW * SENT_PER_W
TOK_PAD = SENT_PAD * L_TOK
CH1_PER_W = SENT_PER_W * L_TOK // 128
ACC1_SL = 328
ACC1_ROWS = NS * ACC1_SL

HEXT = 136
E_PAD = 327680
CH3_PER_W = E_PAD // 128 // NW
ACC3_ROWS = 10112
ACC3_SL = ACC3_ROWS // NS

NB1 = 4


@functools.cache
def _mesh():
    return plsc.VectorSubcoreMesh(core_axis_name="c", subcore_axis_name="s",
                                  num_cores=NC, num_subcores=NS)


def _s1_body(docs_ref, lens_ref, emb_ref, z_ref, out_ref,
             acc, i0, i1, i2, i3, d0, d1, d2, d3, r0, r1, r2, r3,
             lensb, gsem, asem):
    c = lax.axis_index("c")
    s = lax.axis_index("s")
    wid = s * NC + c
    idxb = [i0, i1, i2, i3]
    dstb = [d0, d1, d2, d3]
    rowsb = [r0, r1, r2, r3]

    pltpu.sync_copy(z_ref, acc.at[pl.ds(s * ACC1_SL, ACC1_SL)])
    pltpu.sync_copy(lens_ref.at[pl.ds(wid * SENT_PER_W, SENT_PER_W)], lensb)
    plsc.subcore_barrier()

    tbase = wid * SENT_PER_W * L_TOK
    row0 = s * ACC1_SL

    def fire(ci, b):
        pltpu.sync_copy(docs_ref.at[pl.ds(tbase + ci * 128, 128)], idxb[b])
        pltpu.async_copy(emb_ref.at[idxb[b]], rowsb[b], gsem)

    def drain_scatter():
        pltpu.make_async_copy(emb_ref.at[pl.ds(0, 128)], r0, asem).wait()

    def compute_dst(ci, b):
        for g in range(8):
            lane = lax.iota(jnp.int32, 16)
            q = ci * 128 + g * 16 + lane
            r = q // SENT_PER_W
            sl = q - r * SENT_PER_W
            ln = plsc.load_gather(lensb, [sl])
            valid = r < jnp.maximum(ln, 1)
            dstb[b][pl.ds(g * 16, 16)] = row0 + jnp.where(
                valid, sl, SENT_PER_W + (lane & 7))

    for b in range(2):
        fire(b, b)

    @pl.loop(0, CH1_PER_W - 1, step=NB1)
    def _outer(i):
        for b in range(NB1):
            bb = (b + 2) % NB1
            if b >= 2:
                drain_scatter()
            else:
                @pl.when(i > 0)
                def _():
                    drain_scatter()
            if b == 3:
                @pl.when(i != CH1_PER_W - 1 - NB1)
                def _():
                    fire(i + b + 2, bb)
            else:
                fire(i + b + 2, bb)
            pltpu.make_async_copy(emb_ref.at[idxb[b]], rowsb[b], gsem).wait()
            compute_dst(i + b, b)
            pltpu.async_copy(rowsb[b], acc.at[dstb[b]], asem, add=True)

    drain_scatter()
    last = CH1_PER_W - 1
    pltpu.make_async_copy(emb_ref.at[idxb[last % NB1]],
                          rowsb[last % NB1], gsem).wait()
    compute_dst(last, last % NB1)
    pltpu.sync_copy(rowsb[last % NB1], acc.at[dstb[last % NB1]], add=True)
    drain_scatter()

    plsc.subcore_barrier()
    pltpu.sync_copy(acc.at[pl.ds(s * ACC1_SL, ACC1_SL)],
                    out_ref.at[c, pl.ds(s * ACC1_SL, ACC1_SL)])


@functools.cache
def _stage1():
    return pl.kernel(
        _s1_body,
        out_type=jax.ShapeDtypeStruct((NC, ACC1_ROWS, EMB), jnp.float32),
        mesh=_mesh(),
        compiler_params=pltpu.CompilerParams(
            use_tc_tiling_on_sc=False, needs_layout_passes=False),
        scratch_types=(
            [pltpu.VMEM_SHARED((ACC1_ROWS, EMB), jnp.float32)]
            + [pltpu.VMEM((128,), jnp.int32) for _ in range(NB1)]
            + [pltpu.VMEM((128,), jnp.int32) for _ in range(NB1)]
            + [pltpu.VMEM((128, EMB), jnp.float32) for _ in range(NB1)]
            + [pltpu.VMEM((SENT_PER_W,), jnp.int32),
               pltpu.SemaphoreType.DMA,
               pltpu.SemaphoreType.DMA]),
    )


def _s2_body(sums_ref, lens_ref, Wt_ref, bt_ref, fl_ref, te_ref, Wp_ref,
             Wg_ref, asrc_ref, adst_ref, X_ref, hext_ref, dv_ref):
    f32 = jnp.float32
    s0 = sums_ref[0].reshape(NS, ACC1_SL, EMB)[:, :SENT_PER_W, :]
    s1 = sums_ref[1].reshape(NS, ACC1_SL, EMB)[:, :SENT_PER_W, :]
    sums = jnp.stack([s0, s1], axis=1).reshape(SENT_PAD, EMB)[:N_SENT]
    lens = jnp.maximum(lens_ref[...].astype(f32), 1.0)
    pooled = sums / lens
    d = jnp.maximum(
        jnp.dot(pooled, Wt_ref[...], preferred_element_type=f32)
        + bt_ref[...], 0.0)
    P = (fl_ref[...] ==
         lax.broadcasted_iota(jnp.int32, (N_TOPIC, N_TOPIC), 1)).astype(f32)
    t = jnp.dot(jnp.dot(P, te_ref[...], preferred_element_type=f32),
                Wp_ref[...], preferred_element_type=f32)
    X = jnp.concatenate([d, t], axis=0)
    h = jnp.dot(X, Wg_ref[...], preferred_element_type=f32)
    sc = jnp.dot(h, asrc_ref[...], preferred_element_type=f32)
    dv = jnp.dot(h, adst_ref[...], preferred_element_type=f32)
    X_ref[...] = X
    dv_ref[...] = jnp.pad(dv, ((0, ACC3_ROWS - N_NODES), (0, 0)))
    zcol = jnp.zeros((N_NODES, 1), f32)
    zpad = jnp.zeros((N_NODES, HEXT - EMB - 3), f32)
    hext_ref[...] = jnp.concatenate([h, zcol, sc, dv, zpad], axis=1)


_stage2 = pl.pallas_call(
    _s2_body,
    out_shape=[
        jax.ShapeDtypeStruct((N_NODES, EMB), jnp.float32),
        jax.ShapeDtypeStruct((N_NODES, HEXT), jnp.float32),
        jax.ShapeDtypeStruct((ACC3_ROWS, 1), jnp.float32),
    ],
)


def _s3_body(src_ref, dst_ref, hext_ref, dv_ref, z_ref, out_ref,
             acc, s0, s1, d0, d1, r0, r1, v0, v1, exb, gsem, dsem, asem):
    c = lax.axis_index("c")
    s = lax.axis_index("s")
    wid = s * NC + c
    sidxb = [s0, s1]
    didxb = [d0, d1]
    rowsb = [r0, r1]
    dvb = [v0, v1]

    pltpu.sync_copy(z_ref, acc.at[pl.ds(s * ACC3_SL, ACC3_SL)])
    plsc.subcore_barrier()

    cbase = wid * CH3_PER_W

    def fire(ci, b):
        off = (cbase + ci) * 128
        pltpu.sync_copy(src_ref.at[pl.ds(off, 128)], sidxb[b])
        pltpu.sync_copy(dst_ref.at[pl.ds(off, 128)], didxb[b])
        pltpu.async_copy(hext_ref.at[sidxb[b]], rowsb[b], gsem)
        pltpu.async_copy(dv_ref.at[didxb[b]], dvb[b], dsem)

    def drain_scatter():
        pltpu.make_async_copy(hext_ref.at[pl.ds(0, 128)], r0, asem).wait()

    fire(0, 0)

    @pl.loop(0, CH3_PER_W, step=2)
    def _outer(i):
        for b in range(2):
            if b == 0:
                @pl.when(i > 0)
                def _():
                    drain_scatter()
                fire(i + 1, 1)
            else:
                drain_scatter()

                @pl.when(i != CH3_PER_W - 2)
                def _():
                    fire(i + 2, 0)
            pltpu.make_async_copy(hext_ref.at[sidxb[b]], rowsb[b], gsem).wait()
            pltpu.make_async_copy(dv_ref.at[didxb[b]], dvb[b], dsem).wait()
            rb = rowsb[b]
            for g in range(8):
                ii = lax.iota(jnp.int32, 16) + g * 16
                ssrc = plsc.load_gather(rb, [ii, jnp.full((16,), EMB + 1,
                                                          jnp.int32)])
                e = ssrc + dvb[b][pl.ds(g * 16, 16)]
                e = jnp.maximum(e, 0.2 * e)
                ex = jnp.exp(e)
                exb[pl.ds(g * 16, 16)] = ex
                plsc.store_scatter(rb, [ii, jnp.full((16,), EMB, jnp.int32)],
                                   ex)

            @pl.loop(0, 128)
            def _edge(ei):
                sp = plsc.load_gather(exb, [jnp.full((16,), ei, jnp.int32)])
                for cc in range(EMB // 16):
                    rb[ei, pl.ds(cc * 16, 16)] = rb[ei, pl.ds(cc * 16, 16)] * sp

            pltpu.async_copy(rowsb[b], acc.at[didxb[b]], asem, add=True)

    drain_scatter()

    plsc.subcore_barrier()
    pltpu.sync_copy(acc.at[pl.ds(s * ACC3_SL, ACC3_SL)],
                    out_ref.at[c, pl.ds(s * ACC3_SL, ACC3_SL)])


@functools.cache
def _stage3():
    return pl.kernel(
        _s3_body,
        out_type=jax.ShapeDtypeStruct((NC, ACC3_ROWS, HEXT), jnp.float32),
        mesh=_mesh(),
        compiler_params=pltpu.CompilerParams(
            use_tc_tiling_on_sc=False, needs_layout_passes=False),
        scratch_types=(
            [pltpu.VMEM_SHARED((ACC3_ROWS, HEXT), jnp.float32)]
            + [pltpu.VMEM((128,), jnp.int32) for _ in range(2)]
            + [pltpu.VMEM((128,), jnp.int32) for _ in range(2)]
            + [pltpu.VMEM((128, HEXT), jnp.float32) for _ in range(2)]
            + [pltpu.VMEM((128,), jnp.float32) for _ in range(2)]
            + [pltpu.VMEM((128,), jnp.float32),
               pltpu.SemaphoreType.DMA,
               pltpu.SemaphoreType.DMA,
               pltpu.SemaphoreType.DMA]),
    )


def _s4_body(msg_ref, X_ref, spd_ref, Wc_ref, bc_ref, out_ref):
    f32 = jnp.float32
    m = msg_ref[0] + msg_ref[1]
    msg = m[:N_NODES, :EMB]
    den = m[:N_NODES, EMB:EMB + 1]
    o = msg / jnp.maximum(den, 1e-9)
    o = jnp.where(o > 0, o, jnp.exp(o) - 1.0) + X_ref[...]
    Xs = o[:N_SENT]
    oh = (spd_ref[...] ==
          lax.broadcasted_iota(jnp.int32, (N_SENT, NDOCS), 1)).astype(f32)
    sums = lax.dot_general(oh, Xs, (((0,), (0,)), ((), ())),
                           preferred_element_type=f32)
    cnts = jnp.sum(oh, axis=0)[:, None]
    Xd = sums / jnp.maximum(cnts, 1.0)
    logits = jnp.dot(Xd, Wc_ref[...], preferred_element_type=f32) + bc_ref[...]
    out_ref[...] = jax.nn.softmax(logits, axis=1)


_stage4 = pl.pallas_call(
    _s4_body,
    out_shape=jax.ShapeDtypeStruct((NDOCS, 5), jnp.float32),
)


@jax.jit
def kernel(documents, ent_desc, doc_lens, ent_lens, adj_lists, feature_lists,
           sentPerDoc, word_emb, topic_emb, W_text, b_text, W_topic, W_gat,
           a_src, a_dst, W_cls, b_cls):
    i32 = jnp.int32
    f32 = jnp.float32
    docs_t = (jnp.pad(documents.astype(i32), ((0, SENT_PAD - N_SENT), (0, 0)))
              .reshape(NW, SENT_PER_W, L_TOK)
              .transpose(0, 2, 1).reshape(-1))
    lens_pad = jnp.pad(doc_lens.astype(i32), (0, SENT_PAD - N_SENT))
    z1 = jnp.zeros((ACC1_SL, EMB), f32)
    sums = _stage1()(docs_t, lens_pad, word_emb.astype(f32), z1)

    X, hext, dv = _stage2(
        sums, doc_lens.astype(i32).reshape(-1, 1), W_text.astype(f32),
        b_text.astype(f32).reshape(1, EMB), feature_lists.astype(i32).reshape(-1, 1),
        topic_emb.astype(f32), W_topic.astype(f32), W_gat.astype(f32),
        a_src.astype(f32).reshape(EMB, 1), a_dst.astype(f32).reshape(EMB, 1))

    src = jnp.pad(adj_lists[0].astype(i32), (0, E_PAD - N_EDGES))
    dst = jnp.concatenate([adj_lists[1].astype(i32),
                           N_NODES + (jnp.arange(E_PAD - N_EDGES, dtype=i32)
                                      % (ACC3_ROWS - N_NODES))])
    z3 = jnp.zeros((ACC3_SL, HEXT), f32)
    msg = _stage3()(src, dst, hext, dv.reshape(-1), z3)

    return _stage4(msg, X, sentPerDoc.astype(i32).reshape(-1, 1),
                   W_cls.astype(f32), b_cls.astype(f32).reshape(1, 5))
